# baseline (speedup 1.0000x reference)
import functools

import jax
import jax.numpy as jnp
from jax import lax
from jax.experimental import pallas as pl
from jax.experimental.pallas import tpu as pltpu
from jax.experimental.pallas import tpu_sc as plsc

_HEADS = 4
_K_DIM = 512
_V_DIM = 512
_SUBSIZE = 512
_TOPK = 8
_HALF = _K_DIM // 2
_NEG = -1e30

_NC = 2
_NS = 16
_NW = _NC * _NS
_LANES = 16


_BIG = 1e9


def _topk8(s):
    iota = lax.broadcasted_iota(jnp.int32, s.shape, 1).astype(jnp.float32)
    work = s
    vals, idxs = [], []
    for _ in range(_TOPK):
        m = jnp.max(work, axis=1, keepdims=True)
        z = jnp.where(work == m, iota, _BIG)
        ii = jnp.min(z, axis=1, keepdims=True)
        vals.append(m)
        idxs.append(ii)
        work = jnp.where(z == ii, _NEG, work)
    return jnp.concatenate(vals, axis=1), jnp.concatenate(idxs, axis=1)


def _topk8_wide(s):
    tb, n = s.shape
    ncol = n // 128
    assert ncol == 4
    lane = lax.broadcasted_iota(jnp.int32, (tb, 128), 1).astype(jnp.float32)
    c = [s[:, v * 128 : (v + 1) * 128] for v in range(ncol)]
    ci = [lane + float(v * 128) for v in range(ncol)]

    def comp(av, ai, bv, bi):
        sw = (bv > av) | ((bv == av) & (bi < ai))
        return (
            jnp.where(sw, bv, av),
            jnp.where(sw, bi, ai),
            jnp.where(sw, av, bv),
            jnp.where(sw, ai, bi),
        )

    c[0], ci[0], c[1], ci[1] = comp(c[0], ci[0], c[1], ci[1])
    c[2], ci[2], c[3], ci[3] = comp(c[2], ci[2], c[3], ci[3])
    c[0], ci[0], c[2], ci[2] = comp(c[0], ci[0], c[2], ci[2])
    c[1], ci[1], c[3], ci[3] = comp(c[1], ci[1], c[3], ci[3])
    c[1], ci[1], c[2], ci[2] = comp(c[1], ci[1], c[2], ci[2])

    vals, idxs = [], []
    for _ in range(_TOPK):
        m = jnp.max(c[0], axis=1, keepdims=True)
        z = jnp.where(c[0] == m, ci[0], _BIG)
        ii = jnp.min(z, axis=1, keepdims=True)
        ml = z == ii
        vals.append(m)
        idxs.append(ii)
        c[0] = jnp.where(ml, c[1], c[0])
        ci[0] = jnp.where(ml, ci[1], ci[0])
        c[1] = jnp.where(ml, c[2], c[1])
        ci[1] = jnp.where(ml, ci[2], ci[1])
        c[2] = jnp.where(ml, c[3], c[2])
        ci[2] = jnp.where(ml, ci[3], ci[2])
        c[3] = jnp.where(ml, _NEG, c[3])
    return jnp.concatenate(vals, axis=1), jnp.concatenate(idxs, axis=1)


def _take8(src, j):
    out = jnp.zeros(j.shape, jnp.float32)
    for k in range(_TOPK):
        out = jnp.where(j == float(k), src[:, k : k + 1], out)
    return out


def _tc_body(q_ref, k_ref, sc_ref, nm_ref, ix_ref):
    all_scores = []
    all_idx = []
    for h in range(_HEADS):
        q1 = q_ref[:, h * _K_DIM : h * _K_DIM + _HALF]
        q2 = q_ref[:, h * _K_DIM + _HALF : (h + 1) * _K_DIM]
        k1 = k_ref[h * _SUBSIZE : (h + 1) * _SUBSIZE, :]
        k2 = k_ref[(_HEADS + h) * _SUBSIZE : (_HEADS + h + 1) * _SUBSIZE, :]
        dn = (((1,), (1,)), ((), ()))
        s1 = lax.dot_general(q1, k1, dn, preferred_element_type=jnp.float32)
        s2 = lax.dot_general(q2, k2, dn, preferred_element_type=jnp.float32)
        v1, i1 = _topk8_wide(s1)
        v2, i2 = _topk8_wide(s2)
        comb = jnp.concatenate(
            [v1[:, i : i + 1] + v2 for i in range(_TOPK)], axis=1
        )
        vb, ib = _topk8(comb)
        j1 = jnp.floor(ib * (1.0 / _TOPK))
        j2 = ib - j1 * _TOPK
        idx1 = _take8(i1, j1)
        idx2 = _take8(i2, j2)
        all_scores.append(vb)
        all_idx.append(idx1 * _SUBSIZE + idx2)
    sc = jnp.concatenate(all_scores, axis=1)
    ix = jnp.concatenate(all_idx, axis=1).astype(jnp.int32)
    m = jnp.max(sc, axis=1, keepdims=True)
    e = jnp.exp(sc - m)
    nm = e / jnp.sum(e, axis=1, keepdims=True)
    sc_ref[...] = sc
    nm_ref[...] = nm
    ix_ref[...] = ix


def _tc_call(q, keys, tb=256):
    bt, qd = q.shape
    grid = (bt // tb,)
    ow = _HEADS * _TOPK
    return pl.pallas_call(
        _tc_body,
        grid=grid,
        in_specs=[
            pl.BlockSpec((tb, qd), lambda i: (i, 0)),
            pl.BlockSpec(keys.shape, lambda i: (0, 0)),
        ],
        out_specs=[
            pl.BlockSpec((tb, ow), lambda i: (i, 0)),
            pl.BlockSpec((tb, ow), lambda i: (i, 0)),
            pl.BlockSpec((tb, ow), lambda i: (i, 0)),
        ],
        out_shape=[
            jax.ShapeDtypeStruct((bt, ow), jnp.float32),
            jax.ShapeDtypeStruct((bt, ow), jnp.float32),
            jax.ShapeDtypeStruct((bt, ow), jnp.int32),
        ],
    )(q, keys)


_NRING = 4


def _make_sc_kernel(bt, d):
    kpt = _HEADS * _TOPK
    tpw = bt // _NW
    mesh = plsc.VectorSubcoreMesh(
        core_axis_name="c", subcore_axis_name="s", num_cores=_NC, num_subcores=_NS
    )

    @functools.partial(
        pl.kernel,
        out_type=jax.ShapeDtypeStruct((bt, d), jnp.float32),
        mesh=mesh,
        scratch_types=[
            pltpu.VMEM((tpw * kpt,), jnp.int32),
            pltpu.VMEM((tpw * kpt,), jnp.float32),
            pltpu.VMEM((_NRING, kpt, d), jnp.float32),
            pltpu.VMEM((_NRING, 1, d), jnp.float32),
            [pltpu.SemaphoreType.DMA] * _NRING,
            [pltpu.SemaphoreType.DMA] * _NRING,
        ],
    )
    def sc_kernel(
        values_hbm, idx_hbm, w_hbm, out_hbm, idx_v, w_v, rows_v, acc_v, gsems, osems
    ):
        wid = lax.axis_index("s") * _NC + lax.axis_index("c")
        ebase = wid * (tpw * kpt)
        pltpu.sync_copy(idx_hbm.at[pl.ds(ebase, tpw * kpt)], idx_v)
        pltpu.sync_copy(w_hbm.at[pl.ds(ebase, tpw * kpt)], w_v)

        def start_gather(t, b):
            pltpu.async_copy(
                values_hbm.at[idx_v.at[pl.ds(t * kpt, kpt)]], rows_v.at[b], gsems[b]
            )

        def wait_gather(b):
            pltpu.make_async_copy(
                values_hbm.at[pl.ds(0, kpt)], rows_v.at[b], gsems[b]
            ).wait()

        def start_out(t, b):
            pltpu.async_copy(
                acc_v.at[b], out_hbm.at[pl.ds(wid * tpw + t, 1)], osems[b]
            )

        def wait_out(b):
            pltpu.make_async_copy(
                acc_v.at[b], out_hbm.at[pl.ds(0, 1)], osems[b]
            ).wait()

        for b in range(_NRING):
            start_gather(b, b)

        gdn = lax.GatherDimensionNumbers(
            offset_dims=(), collapsed_slice_dims=(0,), start_index_map=(0,)
        )

        def body(tt, carry):
            for b in range(_NRING):
                t = tt * _NRING + b
                wait_gather(b)

                @pl.when(tt > 0)
                def _():
                    wait_out(b)

                wvecs = [
                    w_v[pl.ds(t * kpt + g * _LANES, _LANES)]
                    for g in range(kpt // _LANES)
                ]
                ws = [
                    lax.gather(
                        wvecs[k // _LANES],
                        jnp.full((_LANES, 1), k % _LANES, jnp.int32),
                        gdn,
                        slice_sizes=(1,),
                        mode=lax.GatherScatterMode.PROMISE_IN_BOUNDS,
                    )
                    for k in range(kpt)
                ]

                def dbody(dd, wc):
                    sl = pl.ds(pl.multiple_of(dd * _LANES, _LANES), _LANES)
                    parts = []
                    for g in range(4):
                        a = wc[g] * rows_v[b, g, sl]
                        for k in range(g + 4, kpt, 4):
                            a = a + wc[k] * rows_v[b, k, sl]
                        parts.append(a)
                    acc_v[b, 0, sl] = (parts[0] + parts[1]) + (parts[2] + parts[3])
                    return wc

                lax.fori_loop(0, d // _LANES, dbody, tuple(ws))
                start_out(t, b)

                @pl.when(t + _NRING < tpw)
                def _():
                    start_gather(t + _NRING, b)

            return carry

        lax.fori_loop(0, tpw // _NRING, body, 0)
        for b in range(_NRING):
            wait_out(b)

    return sc_kernel


def kernel(query, keys, values):
    b, t, qd = query.shape
    bt = b * t
    q = query.reshape(bt, qd)
    sc32, nm32, ix32 = _tc_call(q, keys)
    retrieved = _make_sc_kernel(bt, values.shape[1])(
        values, ix32.reshape(-1), nm32.reshape(-1)
    )
    scores = sc32.reshape(bt, _HEADS, _TOPK)
    indices = ix32.reshape(bt, _HEADS, _TOPK)
    return retrieved, scores, nm32, indices


# speedup vs baseline: 1.0335x; 1.0335x over previous
import functools

import jax
import jax.numpy as jnp
from jax import lax
from jax.experimental import pallas as pl
from jax.experimental.pallas import tpu as pltpu
from jax.experimental.pallas import tpu_sc as plsc

_HEADS = 4
_K_DIM = 512
_V_DIM = 512
_SUBSIZE = 512
_TOPK = 8
_HALF = _K_DIM // 2
_NEG = -1e30

_NC = 2
_NS = 16
_NW = _NC * _NS
_LANES = 16


_BIG = 1e9


def _topk8(s):
    iota = lax.broadcasted_iota(jnp.int32, s.shape, 1).astype(jnp.float32)
    work = s
    vals, idxs = [], []
    for _ in range(_TOPK):
        m = jnp.max(work, axis=1, keepdims=True)
        z = jnp.where(work == m, iota, _BIG)
        ii = jnp.min(z, axis=1, keepdims=True)
        vals.append(m)
        idxs.append(ii)
        work = jnp.where(z == ii, _NEG, work)
    return jnp.concatenate(vals, axis=1), jnp.concatenate(idxs, axis=1)


def _topk8_wide(s):
    tb, n = s.shape
    ncol = n // 128
    assert ncol == 4
    lane = lax.broadcasted_iota(jnp.int32, (tb, 128), 1).astype(jnp.float32)
    c = [s[:, v * 128 : (v + 1) * 128] for v in range(ncol)]
    ci = [lane + float(v * 128) for v in range(ncol)]

    def comp(av, ai, bv, bi):
        sw = (bv > av) | ((bv == av) & (bi < ai))
        return (
            jnp.where(sw, bv, av),
            jnp.where(sw, bi, ai),
            jnp.where(sw, av, bv),
            jnp.where(sw, ai, bi),
        )

    c[0], ci[0], c[1], ci[1] = comp(c[0], ci[0], c[1], ci[1])
    c[2], ci[2], c[3], ci[3] = comp(c[2], ci[2], c[3], ci[3])
    c[0], ci[0], c[2], ci[2] = comp(c[0], ci[0], c[2], ci[2])
    c[1], ci[1], c[3], ci[3] = comp(c[1], ci[1], c[3], ci[3])
    c[1], ci[1], c[2], ci[2] = comp(c[1], ci[1], c[2], ci[2])

    vals, idxs = [], []
    for _ in range(_TOPK):
        m = jnp.max(c[0], axis=1, keepdims=True)
        z = jnp.where(c[0] == m, ci[0], _BIG)
        ii = jnp.min(z, axis=1, keepdims=True)
        ml = z == ii
        vals.append(m)
        idxs.append(ii)
        c[0] = jnp.where(ml, c[1], c[0])
        ci[0] = jnp.where(ml, ci[1], ci[0])
        c[1] = jnp.where(ml, c[2], c[1])
        ci[1] = jnp.where(ml, ci[2], ci[1])
        c[2] = jnp.where(ml, c[3], c[2])
        ci[2] = jnp.where(ml, ci[3], ci[2])
        c[3] = jnp.where(ml, _NEG, c[3])
    return jnp.concatenate(vals, axis=1), jnp.concatenate(idxs, axis=1)


def _take8(src, j):
    out = jnp.zeros(j.shape, jnp.float32)
    for k in range(_TOPK):
        out = jnp.where(j == float(k), src[:, k : k + 1], out)
    return out


def _tc_body(q_ref, k_ref, sc_ref, nm_ref, ix_ref):
    all_scores = []
    all_idx = []
    for h in range(_HEADS):
        q1 = q_ref[:, h * _K_DIM : h * _K_DIM + _HALF]
        q2 = q_ref[:, h * _K_DIM + _HALF : (h + 1) * _K_DIM]
        k1 = k_ref[h * _SUBSIZE : (h + 1) * _SUBSIZE, :]
        k2 = k_ref[(_HEADS + h) * _SUBSIZE : (_HEADS + h + 1) * _SUBSIZE, :]
        dn = (((1,), (1,)), ((), ()))
        s1 = lax.dot_general(q1, k1, dn, preferred_element_type=jnp.float32)
        s2 = lax.dot_general(q2, k2, dn, preferred_element_type=jnp.float32)
        v1, i1 = _topk8(s1)
        v2, i2 = _topk8(s2)
        comb = jnp.concatenate(
            [v1[:, i : i + 1] + v2 for i in range(_TOPK)], axis=1
        )
        vb, ib = _topk8(comb)
        j1 = jnp.floor(ib * (1.0 / _TOPK))
        j2 = ib - j1 * _TOPK
        idx1 = _take8(i1, j1)
        idx2 = _take8(i2, j2)
        all_scores.append(vb)
        all_idx.append(idx1 * _SUBSIZE + idx2)
    sc = jnp.concatenate(all_scores, axis=1)
    ix = jnp.concatenate(all_idx, axis=1).astype(jnp.int32)
    m = jnp.max(sc, axis=1, keepdims=True)
    e = jnp.exp(sc - m)
    nm = e / jnp.sum(e, axis=1, keepdims=True)
    sc_ref[...] = sc
    nm_ref[...] = nm
    ix_ref[...] = ix


def _tc_call(q, keys, tb=256):
    bt, qd = q.shape
    grid = (bt // tb,)
    ow = _HEADS * _TOPK
    return pl.pallas_call(
        _tc_body,
        grid=grid,
        in_specs=[
            pl.BlockSpec((tb, qd), lambda i: (i, 0)),
            pl.BlockSpec(keys.shape, lambda i: (0, 0)),
        ],
        out_specs=[
            pl.BlockSpec((tb, ow), lambda i: (i, 0)),
            pl.BlockSpec((tb, ow), lambda i: (i, 0)),
            pl.BlockSpec((tb, ow), lambda i: (i, 0)),
        ],
        out_shape=[
            jax.ShapeDtypeStruct((bt, ow), jnp.float32),
            jax.ShapeDtypeStruct((bt, ow), jnp.float32),
            jax.ShapeDtypeStruct((bt, ow), jnp.int32),
        ],
    )(q, keys)


_NRING = 4


def _make_sc_kernel(bt, d):
    kpt = _HEADS * _TOPK
    tpw = bt // _NW
    mesh = plsc.VectorSubcoreMesh(
        core_axis_name="c", subcore_axis_name="s", num_cores=_NC, num_subcores=_NS
    )

    @functools.partial(
        pl.kernel,
        out_type=jax.ShapeDtypeStruct((bt, d), jnp.float32),
        mesh=mesh,
        scratch_types=[
            pltpu.VMEM((tpw * kpt,), jnp.int32),
            pltpu.VMEM((tpw * kpt,), jnp.float32),
            pltpu.VMEM((_NRING, kpt, d), jnp.float32),
            pltpu.VMEM((_NRING, 1, d), jnp.float32),
            [pltpu.SemaphoreType.DMA] * _NRING,
            [pltpu.SemaphoreType.DMA] * _NRING,
        ],
    )
    def sc_kernel(
        values_hbm, idx_hbm, w_hbm, out_hbm, idx_v, w_v, rows_v, acc_v, gsems, osems
    ):
        wid = lax.axis_index("s") * _NC + lax.axis_index("c")
        ebase = wid * (tpw * kpt)
        pltpu.sync_copy(idx_hbm.at[pl.ds(ebase, tpw * kpt)], idx_v)
        pltpu.sync_copy(w_hbm.at[pl.ds(ebase, tpw * kpt)], w_v)

        def start_gather(t, b):
            pltpu.async_copy(
                values_hbm.at[idx_v.at[pl.ds(t * kpt, kpt)]], rows_v.at[b], gsems[b]
            )

        def wait_gather(b):
            pltpu.make_async_copy(
                values_hbm.at[pl.ds(0, kpt)], rows_v.at[b], gsems[b]
            ).wait()

        def start_out(t, b):
            pltpu.async_copy(
                acc_v.at[b], out_hbm.at[pl.ds(wid * tpw + t, 1)], osems[b]
            )

        def wait_out(b):
            pltpu.make_async_copy(
                acc_v.at[b], out_hbm.at[pl.ds(0, 1)], osems[b]
            ).wait()

        for b in range(_NRING):
            start_gather(b, b)

        gdn = lax.GatherDimensionNumbers(
            offset_dims=(), collapsed_slice_dims=(0,), start_index_map=(0,)
        )

        def body(tt, carry):
            for b in range(_NRING):
                t = tt * _NRING + b
                wait_gather(b)

                @pl.when(tt > 0)
                def _():
                    wait_out(b)

                wvecs = [
                    w_v[pl.ds(t * kpt + g * _LANES, _LANES)]
                    for g in range(kpt // _LANES)
                ]
                ws = [
                    lax.gather(
                        wvecs[k // _LANES],
                        jnp.full((_LANES, 1), k % _LANES, jnp.int32),
                        gdn,
                        slice_sizes=(1,),
                        mode=lax.GatherScatterMode.PROMISE_IN_BOUNDS,
                    )
                    for k in range(kpt)
                ]

                def dbody(dd, wc):
                    sl = pl.ds(pl.multiple_of(dd * _LANES, _LANES), _LANES)
                    parts = []
                    for g in range(4):
                        a = wc[g] * rows_v[b, g, sl]
                        for k in range(g + 4, kpt, 4):
                            a = a + wc[k] * rows_v[b, k, sl]
                        parts.append(a)
                    acc_v[b, 0, sl] = (parts[0] + parts[1]) + (parts[2] + parts[3])
                    return wc

                lax.fori_loop(0, d // _LANES, dbody, tuple(ws))
                start_out(t, b)

                @pl.when(t + _NRING < tpw)
                def _():
                    start_gather(t + _NRING, b)

            return carry

        lax.fori_loop(0, tpw // _NRING, body, 0)
        for b in range(_NRING):
            wait_out(b)

    return sc_kernel


def kernel(query, keys, values):
    b, t, qd = query.shape
    bt = b * t
    q = query.reshape(bt, qd)
    half = bt // 2
    sc_k = _make_sc_kernel(half, values.shape[1])
    sc_a, nm_a, ix_a = _tc_call(q[:half], keys)
    ret_a = sc_k(values, ix_a.reshape(-1), nm_a.reshape(-1))
    sc_b, nm_b, ix_b = _tc_call(q[half:], keys)
    ret_b = sc_k(values, ix_b.reshape(-1), nm_b.reshape(-1))
    retrieved = jnp.concatenate([ret_a, ret_b], axis=0)
    sc32 = jnp.concatenate([sc_a, sc_b], axis=0)
    nm32 = jnp.concatenate([nm_a, nm_b], axis=0)
    ix32 = jnp.concatenate([ix_a, ix_b], axis=0)
    scores = sc32.reshape(bt, _HEADS, _TOPK)
    indices = ix32.reshape(bt, _HEADS, _TOPK)
    return retrieved, scores, nm32, indices


# speedup vs baseline: 1.0479x; 1.0139x over previous
import functools

import jax
import jax.numpy as jnp
from jax import lax
from jax.experimental import pallas as pl
from jax.experimental.pallas import tpu as pltpu
from jax.experimental.pallas import tpu_sc as plsc

_HEADS = 4
_K_DIM = 512
_V_DIM = 512
_SUBSIZE = 512
_TOPK = 8
_HALF = _K_DIM // 2
_NEG = -1e30

_NC = 2
_NS = 16
_NW = _NC * _NS
_LANES = 16


_BIG = 1e9


def _topk8(s):
    iota = lax.broadcasted_iota(jnp.int32, s.shape, 1).astype(jnp.float32)
    work = s
    vals, idxs = [], []
    for _ in range(_TOPK):
        m = jnp.max(work, axis=1, keepdims=True)
        z = jnp.where(work == m, iota, _BIG)
        ii = jnp.min(z, axis=1, keepdims=True)
        vals.append(m)
        idxs.append(ii)
        work = jnp.where(z == ii, _NEG, work)
    return jnp.concatenate(vals, axis=1), jnp.concatenate(idxs, axis=1)


def _topk8_wide(s):
    tb, n = s.shape
    ncol = n // 128
    assert ncol == 4
    lane = lax.broadcasted_iota(jnp.int32, (tb, 128), 1).astype(jnp.float32)
    c = [s[:, v * 128 : (v + 1) * 128] for v in range(ncol)]
    ci = [lane + float(v * 128) for v in range(ncol)]

    def comp(av, ai, bv, bi):
        sw = (bv > av) | ((bv == av) & (bi < ai))
        return (
            jnp.where(sw, bv, av),
            jnp.where(sw, bi, ai),
            jnp.where(sw, av, bv),
            jnp.where(sw, ai, bi),
        )

    c[0], ci[0], c[1], ci[1] = comp(c[0], ci[0], c[1], ci[1])
    c[2], ci[2], c[3], ci[3] = comp(c[2], ci[2], c[3], ci[3])
    c[0], ci[0], c[2], ci[2] = comp(c[0], ci[0], c[2], ci[2])
    c[1], ci[1], c[3], ci[3] = comp(c[1], ci[1], c[3], ci[3])
    c[1], ci[1], c[2], ci[2] = comp(c[1], ci[1], c[2], ci[2])

    vals, idxs = [], []
    for _ in range(_TOPK):
        m = jnp.max(c[0], axis=1, keepdims=True)
        z = jnp.where(c[0] == m, ci[0], _BIG)
        ii = jnp.min(z, axis=1, keepdims=True)
        ml = z == ii
        vals.append(m)
        idxs.append(ii)
        c[0] = jnp.where(ml, c[1], c[0])
        ci[0] = jnp.where(ml, ci[1], ci[0])
        c[1] = jnp.where(ml, c[2], c[1])
        ci[1] = jnp.where(ml, ci[2], ci[1])
        c[2] = jnp.where(ml, c[3], c[2])
        ci[2] = jnp.where(ml, ci[3], ci[2])
        c[3] = jnp.where(ml, _NEG, c[3])
    return jnp.concatenate(vals, axis=1), jnp.concatenate(idxs, axis=1)


def _take8(src, j):
    out = jnp.zeros(j.shape, jnp.float32)
    for k in range(_TOPK):
        out = jnp.where(j == float(k), src[:, k : k + 1], out)
    return out


def _tc_body(q_ref, k_ref, sc_ref, nm_ref, ix_ref):
    all_scores = []
    all_idx = []
    for h in range(_HEADS):
        q1 = q_ref[:, h * _K_DIM : h * _K_DIM + _HALF]
        q2 = q_ref[:, h * _K_DIM + _HALF : (h + 1) * _K_DIM]
        k1 = k_ref[h * _SUBSIZE : (h + 1) * _SUBSIZE, :]
        k2 = k_ref[(_HEADS + h) * _SUBSIZE : (_HEADS + h + 1) * _SUBSIZE, :]
        dn = (((1,), (1,)), ((), ()))
        s1 = lax.dot_general(q1, k1, dn, preferred_element_type=jnp.float32)
        s2 = lax.dot_general(q2, k2, dn, preferred_element_type=jnp.float32)
        v1, i1 = _topk8(s1)
        v2, i2 = _topk8(s2)
        comb = jnp.concatenate(
            [v1[:, i : i + 1] + v2 for i in range(_TOPK)], axis=1
        )
        vb, ib = _topk8(comb)
        j1 = jnp.floor(ib * (1.0 / _TOPK))
        j2 = ib - j1 * _TOPK
        idx1 = _take8(i1, j1)
        idx2 = _take8(i2, j2)
        all_scores.append(vb)
        all_idx.append(idx1 * _SUBSIZE + idx2)
    sc = jnp.concatenate(all_scores, axis=1)
    ix = jnp.concatenate(all_idx, axis=1).astype(jnp.int32)
    m = jnp.max(sc, axis=1, keepdims=True)
    e = jnp.exp(sc - m)
    nm = e / jnp.sum(e, axis=1, keepdims=True)
    sc_ref[...] = sc
    nm_ref[...] = nm
    ix_ref[...] = ix


def _tc_call(q, keys, tb=256):
    bt, qd = q.shape
    grid = (bt // tb,)
    ow = _HEADS * _TOPK
    return pl.pallas_call(
        _tc_body,
        grid=grid,
        in_specs=[
            pl.BlockSpec((tb, qd), lambda i: (i, 0)),
            pl.BlockSpec(keys.shape, lambda i: (0, 0)),
        ],
        out_specs=[
            pl.BlockSpec((tb, ow), lambda i: (i, 0)),
            pl.BlockSpec((tb, ow), lambda i: (i, 0)),
            pl.BlockSpec((tb, ow), lambda i: (i, 0)),
        ],
        out_shape=[
            jax.ShapeDtypeStruct((bt, ow), jnp.float32),
            jax.ShapeDtypeStruct((bt, ow), jnp.float32),
            jax.ShapeDtypeStruct((bt, ow), jnp.int32),
        ],
    )(q, keys)


_NRING = 4


def _make_sc_kernel(bt, d):
    kpt = _HEADS * _TOPK
    tpw = bt // _NW
    mesh = plsc.VectorSubcoreMesh(
        core_axis_name="c", subcore_axis_name="s", num_cores=_NC, num_subcores=_NS
    )

    @functools.partial(
        pl.kernel,
        out_type=jax.ShapeDtypeStruct((bt, d), jnp.float32),
        mesh=mesh,
        scratch_types=[
            pltpu.VMEM((tpw * kpt,), jnp.int32),
            pltpu.VMEM((tpw * kpt,), jnp.float32),
            pltpu.VMEM((_NRING, kpt, d), jnp.float32),
            pltpu.VMEM((_NRING, 1, d), jnp.float32),
            [pltpu.SemaphoreType.DMA] * _NRING,
            [pltpu.SemaphoreType.DMA] * _NRING,
        ],
    )
    def sc_kernel(
        values_hbm, idx_hbm, w_hbm, out_hbm, idx_v, w_v, rows_v, acc_v, gsems, osems
    ):
        wid = lax.axis_index("s") * _NC + lax.axis_index("c")
        ebase = wid * (tpw * kpt)
        pltpu.sync_copy(idx_hbm.at[pl.ds(ebase, tpw * kpt)], idx_v)
        pltpu.sync_copy(w_hbm.at[pl.ds(ebase, tpw * kpt)], w_v)

        def start_gather(t, b):
            pltpu.async_copy(
                values_hbm.at[idx_v.at[pl.ds(t * kpt, kpt)]], rows_v.at[b], gsems[b]
            )

        def wait_gather(b):
            pltpu.make_async_copy(
                values_hbm.at[pl.ds(0, kpt)], rows_v.at[b], gsems[b]
            ).wait()

        def start_out(t, b):
            pltpu.async_copy(
                acc_v.at[b], out_hbm.at[pl.ds(wid * tpw + t, 1)], osems[b]
            )

        def wait_out(b):
            pltpu.make_async_copy(
                acc_v.at[b], out_hbm.at[pl.ds(0, 1)], osems[b]
            ).wait()

        for b in range(_NRING):
            start_gather(b, b)

        gdn = lax.GatherDimensionNumbers(
            offset_dims=(), collapsed_slice_dims=(0,), start_index_map=(0,)
        )

        def body(tt, carry):
            for b in range(_NRING):
                t = tt * _NRING + b
                wait_gather(b)

                @pl.when(tt > 0)
                def _():
                    wait_out(b)

                wvecs = [
                    w_v[pl.ds(t * kpt + g * _LANES, _LANES)]
                    for g in range(kpt // _LANES)
                ]
                ws = [
                    lax.gather(
                        wvecs[k // _LANES],
                        jnp.full((_LANES, 1), k % _LANES, jnp.int32),
                        gdn,
                        slice_sizes=(1,),
                        mode=lax.GatherScatterMode.PROMISE_IN_BOUNDS,
                    )
                    for k in range(kpt)
                ]

                def dbody(dd, wc):
                    sl = pl.ds(pl.multiple_of(dd * _LANES, _LANES), _LANES)
                    parts = []
                    for g in range(4):
                        a = wc[g] * rows_v[b, g, sl]
                        for k in range(g + 4, kpt, 4):
                            a = a + wc[k] * rows_v[b, k, sl]
                        parts.append(a)
                    acc_v[b, 0, sl] = (parts[0] + parts[1]) + (parts[2] + parts[3])
                    return wc

                lax.fori_loop(0, d // _LANES, dbody, tuple(ws))
                start_out(t, b)

                @pl.when(t + _NRING < tpw)
                def _():
                    start_gather(t + _NRING, b)

            return carry

        lax.fori_loop(0, tpw // _NRING, body, 0)
        for b in range(_NRING):
            wait_out(b)

    return sc_kernel


def kernel(query, keys, values):
    b, t, qd = query.shape
    bt = b * t
    q = query.reshape(bt, qd)
    nsplit = 4
    chunk = bt // nsplit
    sc_k = _make_sc_kernel(chunk, values.shape[1])
    parts = []
    for s in range(nsplit):
        sc_p, nm_p, ix_p = _tc_call(q[s * chunk : (s + 1) * chunk], keys)
        ret_p = sc_k(values, ix_p.reshape(-1), nm_p.reshape(-1))
        parts.append((ret_p, sc_p, nm_p, ix_p))
    retrieved = jnp.concatenate([p[0] for p in parts], axis=0)
    sc32 = jnp.concatenate([p[1] for p in parts], axis=0)
    nm32 = jnp.concatenate([p[2] for p in parts], axis=0)
    ix32 = jnp.concatenate([p[3] for p in parts], axis=0)
    scores = sc32.reshape(bt, _HEADS, _TOPK)
    indices = ix32.reshape(bt, _HEADS, _TOPK)
    return retrieved, scores, nm32, indices
